# Initial kernel scaffold; baseline (speedup 1.0000x reference)
import jax
import jax.numpy as jnp
from jax.experimental import pallas as pl


def kernel(x, w_in, w_out):
    raise NotImplementedError("write your pallas kernel here")


# speedup vs baseline: 1.3113x; 1.3113x over previous
import functools

import jax
import jax.numpy as jnp
from jax import lax
from jax.experimental import pallas as pl
from jax.experimental.pallas import tpu as pltpu
from jax.experimental.pallas import tpu_sc as plsc

_BM = 1024
_BN = 2048
_GW = 128


def _argmin_body(nbn, a_ref, b_ref, x_ref, w_ref, o_ref, min_ref, idx_ref):
    j = pl.program_id(1)

    @pl.when(j == 0)
    def _():
        min_ref[...] = jnp.full(min_ref.shape, jnp.inf, jnp.float32)
        idx_ref[...] = jnp.zeros(idx_ref.shape, jnp.int32)

    c = lax.dot_general(
        x_ref[...], w_ref[...],
        dimension_numbers=(((1,), (1,)), ((), ())),
        preferred_element_type=jnp.float32,
    )
    dist = (a_ref[...] + b_ref[0]) - 2.0 * c
    m = jnp.min(dist, axis=1, keepdims=True)
    iota = lax.broadcasted_iota(jnp.int32, dist.shape, 1)
    bn = dist.shape[1]
    li = jnp.min(jnp.where(dist == m, iota, bn), axis=1, keepdims=True) + j * bn
    better = m < min_ref[...]
    idx_ref[...] = jnp.where(better, li, idx_ref[...])
    min_ref[...] = jnp.where(better, m, min_ref[...])

    @pl.when(j == nbn - 1)
    def _():
        o_ref[...] = idx_ref[...]


def _argmin(xm, w, a, b):
    bm, d = xm.shape
    kc = w.shape[0]
    nbm, nbn = bm // _BM, kc // _BN
    b3 = b.reshape(nbn, 1, _BN)
    return pl.pallas_call(
        functools.partial(_argmin_body, nbn),
        grid=(nbm, nbn),
        in_specs=[
            pl.BlockSpec((_BM, 1), lambda i, j: (i, 0)),
            pl.BlockSpec((1, 1, _BN), lambda i, j: (j, 0, 0)),
            pl.BlockSpec((_BM, d), lambda i, j: (i, 0)),
            pl.BlockSpec((_BN, d), lambda i, j: (j, 0)),
        ],
        out_specs=pl.BlockSpec((_BM, 1), lambda i, j: (i, 0)),
        out_shape=jax.ShapeDtypeStruct((bm, 1), jnp.int32),
        scratch_shapes=[
            pltpu.VMEM((_BM, 1), jnp.float32),
            pltpu.VMEM((_BM, 1), jnp.int32),
        ],
        compiler_params=pltpu.CompilerParams(
            dimension_semantics=("parallel", "arbitrary")),
    )(a, b3, xm, w)


def _sc_gather(table, idx):
    b = idx.shape[1]
    d = table.shape[1]
    mesh = plsc.VectorSubcoreMesh(
        core_axis_name="core", subcore_axis_name="subcore")

    @pl.kernel(out_type=jax.ShapeDtypeStruct((b, d), table.dtype), mesh=mesh)
    def k(tab_hbm, i_hbm, o_hbm):
        def body(i_vmem, o_vmem):
            pltpu.sync_copy(tab_hbm.at[i_vmem.at[0]], o_vmem)

        pltpu.emit_pipeline(
            body,
            grid=(b // _GW,),
            in_specs=[pl.BlockSpec((1, _GW), index_map=lambda i: (0, i))],
            out_specs=[pl.BlockSpec((_GW, d), index_map=lambda i: (i, 0))],
            core_axis_name=("core", "subcore"),
            dimension_semantics=(pltpu.PARALLEL,),
        )(i_hbm, o_hbm)

    return k(table, idx)


def _finalize_body(f_ref, q_ref, r_ref, o_ref, l_ref):
    f = f_ref[...]
    q = q_ref[...]
    r = r_ref[...]
    t = r - f
    o_ref[...] = f + t
    s1 = jnp.sum(t * t)
    d2 = f - q
    s2 = jnp.sum(d2 * d2)
    d3 = q - r
    s3 = jnp.sum(d3 * d3)
    loss = 1.25 * (s1 + s2 + s3) / jnp.float32(f.size)
    l_ref[...] = jnp.full((1, 1), loss, jnp.float32)


def _finalize(flat, xq, rq):
    out, loss = pl.pallas_call(
        _finalize_body,
        out_shape=(
            jax.ShapeDtypeStruct(flat.shape, flat.dtype),
            jax.ShapeDtypeStruct((1, 1), jnp.float32),
        ),
    )(flat, xq, rq)
    return out, loss[0, 0]


def kernel(x, w_in, w_out):
    b = x.shape[0]
    flat = x.reshape(b, -1)
    a1 = jnp.sum(flat ** 2, axis=1, keepdims=True)
    b1 = jnp.sum(w_in ** 2, axis=1)
    i1 = _argmin(flat, w_in, a1, b1)
    xq = _sc_gather(w_in, i1.reshape(1, b))
    a2 = jnp.sum(xq ** 2, axis=1, keepdims=True)
    b2 = jnp.sum(w_out ** 2, axis=1)
    i2 = _argmin(xq, w_out, a2, b2)
    rq = _sc_gather(w_out, i2.reshape(1, b))
    out, loss = _finalize(flat, xq, rq)
    return out.reshape(x.shape), loss
